# baseline (speedup 1.0000x reference)
import jax
import jax.numpy as jnp
from jax import lax
from jax.experimental import pallas as pl
from jax.experimental.pallas import tpu as pltpu
from jax.experimental.pallas import tpu_sc as plsc

N_NODES = 10000
N_EDGES = 320000
NP = 10016
C = 128
NBUF = 4
NCH = 160
STRIPE = NP // 16
NCHUNKS = 16 * NCH
EP = NCHUNKS * C


def _make_sc_seg(D, with_deg):
    mesh = plsc.VectorSubcoreMesh(core_axis_name="c", subcore_axis_name="s")
    outs = [jax.ShapeDtypeStruct((NP, D), jnp.float32)]
    scratch = [
        pltpu.VMEM((NCH, C), jnp.int32),
        pltpu.VMEM((NCH, C), jnp.int32),
        pltpu.VMEM_SHARED((NP, D), jnp.float32),
    ]
    scratch += [pltpu.VMEM((C, D), jnp.float32) for _ in range(NBUF)]
    scratch += [pltpu.SemaphoreType.DMA for _ in range(NBUF)]
    if with_deg:
        outs.append(jax.ShapeDtypeStruct((NP, 16), jnp.float32))
        scratch += [
            pltpu.VMEM((C, 16), jnp.float32),
            pltpu.VMEM_SHARED((NP, 16), jnp.float32),
        ]

    def body(*refs):
        if with_deg:
            (z, srcp, dstp, zacc, zdeg, ones, part, degpart,
             src_all, dst_all, acc_sh, *rest) = refs
            rows = rest[:NBUF]
            sems = rest[NBUF:2 * NBUF]
            ones_v, deg_sh = rest[2 * NBUF:]
        else:
            (z, srcp, dstp, zacc, part,
             src_all, dst_all, acc_sh, *rest) = refs
            rows = rest[:NBUF]
            sems = rest[NBUF:2 * NBUF]
        c = lax.axis_index("c")
        s = lax.axis_index("s")

        @pl.when(c == 0)
        def _work():
            row0 = s * STRIPE
            pltpu.sync_copy(zacc.at[pl.ds(row0, STRIPE)],
                            acc_sh.at[pl.ds(row0, STRIPE)])
            if with_deg:
                pltpu.sync_copy(zdeg.at[pl.ds(row0, STRIPE)],
                                deg_sh.at[pl.ds(row0, STRIPE)])
                pltpu.sync_copy(ones, ones_v)
            pltpu.sync_copy(srcp.at[pl.ds(s * NCH, NCH)], src_all)
            pltpu.sync_copy(dstp.at[pl.ds(s * NCH, NCH)], dst_all)
            plsc.subcore_barrier()

            for b in range(NBUF):
                pltpu.async_copy(z.at[src_all.at[b]], rows[b], sems[b])

            def group(g, carry):
                for b in range(NBUF):
                    i = g * NBUF + b
                    pltpu.make_async_copy(z.at[src_all.at[i]], rows[b],
                                          sems[b]).wait()
                    pltpu.sync_copy(rows[b], acc_sh.at[dst_all.at[i]],
                                    add=True)
                    if with_deg:
                        pltpu.sync_copy(ones_v, deg_sh.at[dst_all.at[i]],
                                        add=True)
                    pltpu.async_copy(z.at[src_all.at[i + NBUF]], rows[b],
                                     sems[b])
                return carry

            lax.fori_loop(0, NCH // NBUF - 1, group, 0)
            for b in range(NBUF):
                i = NCH - NBUF + b
                pltpu.make_async_copy(z.at[src_all.at[i]], rows[b],
                                      sems[b]).wait()
                pltpu.sync_copy(rows[b], acc_sh.at[dst_all.at[i]], add=True)
                if with_deg:
                    pltpu.sync_copy(ones_v, deg_sh.at[dst_all.at[i]],
                                    add=True)
            plsc.subcore_barrier()

            pltpu.sync_copy(acc_sh.at[pl.ds(row0, STRIPE)],
                            part.at[pl.ds(row0, STRIPE)])
            if with_deg:
                pltpu.sync_copy(deg_sh.at[pl.ds(row0, STRIPE)],
                                degpart.at[pl.ds(row0, STRIPE)])

    return pl.kernel(body, out_type=tuple(outs), mesh=mesh,
                     scratch_types=tuple(scratch),
                     compiler_params=pltpu.CompilerParams(
                         use_tc_tiling_on_sc=False,
                         needs_layout_passes=False))


_B = 2000


def _tc1(x, wl, wr, b):
    def body(x_ref, wl_ref, wr_ref, b_ref, z_ref, y_ref):
        xb = x_ref[...]
        z_ref[...] = jnp.dot(xb, wl_ref[...], preferred_element_type=jnp.float32)
        y_ref[...] = (jnp.dot(xb, wr_ref[...], preferred_element_type=jnp.float32)
                      + b_ref[0:1, :])
    full = lambda i: (0, 0)
    row = lambda i: (i, 0)
    return pl.pallas_call(
        body,
        grid=(N_NODES // _B,),
        in_specs=[pl.BlockSpec((_B, 128), row), pl.BlockSpec((128, 64), full),
                  pl.BlockSpec((128, 64), full), pl.BlockSpec((8, 64), full)],
        out_specs=[pl.BlockSpec((_B, 64), row), pl.BlockSpec((_B, 64), row)],
        out_shape=[jax.ShapeDtypeStruct((N_NODES, 64), jnp.float32)] * 2,
    )(x, wl, wr, b)


def _tc2(part1, degpart, y1, wl, wr, b):
    def body(p_ref, d_ref, y1_ref, wl_ref, wr_ref, b_ref, z_ref, y_ref):
        degc = jnp.maximum(d_ref[...][:, 0:1], 1.0)
        h1 = jnp.maximum(p_ref[...] / degc + y1_ref[...], 0.0)
        z_ref[...] = jnp.dot(h1, wl_ref[...], preferred_element_type=jnp.float32)
        y_ref[...] = (jnp.dot(h1, wr_ref[...], preferred_element_type=jnp.float32)
                      + b_ref[0:1, :])
    full = lambda i: (0, 0)
    row = lambda i: (i, 0)
    return pl.pallas_call(
        body,
        grid=(N_NODES // _B,),
        in_specs=[pl.BlockSpec((_B, 64), row), pl.BlockSpec((_B, 16), row),
                  pl.BlockSpec((_B, 64), row), pl.BlockSpec((64, 32), full),
                  pl.BlockSpec((64, 32), full), pl.BlockSpec((8, 32), full)],
        out_specs=[pl.BlockSpec((_B, 32), row), pl.BlockSpec((_B, 32), row)],
        out_shape=[jax.ShapeDtypeStruct((N_NODES, 32), jnp.float32),
                   jax.ShapeDtypeStruct((N_NODES, 32), jnp.float32)],
    )(part1, degpart, y1, wl, wr, b)


def _tc3(part2, degpart, y2, wc1, b1, wc2, b2):
    def body(q_ref, d_ref, y2_ref, wc1_ref, b1_ref, wc2_ref, b2_ref, out_ref):
        degc = jnp.maximum(d_ref[...][:, 0:1], 1.0)
        h2 = jnp.maximum(q_ref[...] / degc + y2_ref[...], 0.0)
        c1 = jnp.maximum(
            jnp.dot(h2, wc1_ref[...], preferred_element_type=jnp.float32)
            + b1_ref[0:1, :], 0.0)
        out_ref[...] = (jnp.dot(c1, wc2_ref[...], preferred_element_type=jnp.float32)
                        + b2_ref[0:1, :])
    full = lambda i: (0, 0)
    row = lambda i: (i, 0)
    return pl.pallas_call(
        body,
        grid=(N_NODES // _B,),
        in_specs=[pl.BlockSpec((_B, 32), row), pl.BlockSpec((_B, 16), row),
                  pl.BlockSpec((_B, 32), row), pl.BlockSpec((32, 16), full),
                  pl.BlockSpec((8, 16), full), pl.BlockSpec((16, 2), full),
                  pl.BlockSpec((8, 2), full)],
        out_specs=pl.BlockSpec((_B, 2), row),
        out_shape=jax.ShapeDtypeStruct((N_NODES, 2), jnp.float32),
    )(part2, degpart, y2, wc1, b1, wc2, b2)


def kernel(x, edge_index, W1l, W1r, b1, W2l, W2r, b2, Wc1, bc1, Wc2, bc2):
    f32 = jnp.float32
    src = edge_index[0].astype(jnp.int32)
    dst = edge_index[1].astype(jnp.int32)
    pad = EP - N_EDGES
    srcp = jnp.concatenate([src, jnp.zeros((pad,), jnp.int32)]
                           ).reshape(NCHUNKS, C)
    dstp = jnp.concatenate([dst, jnp.full((pad,), N_NODES, jnp.int32)]
                           ).reshape(NCHUNKS, C)

    zacc64 = jnp.zeros((NP, 64), f32)
    zacc32 = jnp.zeros((NP, 32), f32)
    zdeg = jnp.zeros((NP, 16), f32)
    ones = jnp.ones((C, 16), f32)

    z1, y1 = _tc1(x, W1l.T, W1r.T, jnp.broadcast_to(b1, (8, 64)))

    part1, degpart = _make_sc_seg(64, True)(z1, srcp, dstp, zacc64, zdeg, ones)

    z2, y2 = _tc2(part1[:N_NODES], degpart[:N_NODES], y1, W2l.T, W2r.T,
                  jnp.broadcast_to(b2, (8, 32)))

    (part2,) = _make_sc_seg(32, False)(z2, srcp, dstp, zacc32)

    out = _tc3(part2[:N_NODES], degpart[:N_NODES], y2,
               Wc1.T, jnp.broadcast_to(bc1, (8, 16)),
               Wc2.T, jnp.broadcast_to(bc2, (8, 2)))
    return out


# speedup vs baseline: 1.3914x; 1.3914x over previous
import jax
import jax.numpy as jnp
from jax import lax
from jax.experimental import pallas as pl
from jax.experimental.pallas import tpu as pltpu
from jax.experimental.pallas import tpu_sc as plsc

N_NODES = 10000
N_EDGES = 320000
NP = 10016
C = 128
NBUF = 5
NCH0 = 145
NCH1 = 15
STRIPE = NP // 16
NCHUNKS = 16 * (NCH0 + NCH1)
EP = NCHUNKS * C
NCPAD = NCHUNKS + (NCH0 - NCH1)


def _make_sc_seg(D, with_deg):
    mesh = plsc.VectorSubcoreMesh(core_axis_name="c", subcore_axis_name="s")
    outs = [jax.ShapeDtypeStruct((2, NP, D), jnp.float32)]
    scratch = [
        pltpu.VMEM((NCH0, C), jnp.int32),
        pltpu.VMEM((NCH0, C), jnp.int32),
        pltpu.VMEM_SHARED((NP, D), jnp.float32),
    ]
    scratch += [pltpu.VMEM((C, D), jnp.float32) for _ in range(NBUF)]
    scratch += [pltpu.SemaphoreType.DMA for _ in range(NBUF)]
    if with_deg:
        outs.append(jax.ShapeDtypeStruct((2, NP, 16), jnp.float32))
        scratch += [
            pltpu.VMEM((C, 16), jnp.float32),
            pltpu.VMEM_SHARED((NP, 16), jnp.float32),
        ]

    def body(*refs):
        if with_deg:
            (z, srcp, dstp, zacc, zdeg, ones, part, degpart,
             src_all, dst_all, acc_sh, *rest) = refs
            rows = rest[:NBUF]
            sems = rest[NBUF:2 * NBUF]
            ones_v, deg_sh = rest[2 * NBUF:]
        else:
            (z, srcp, dstp, zacc, part,
             src_all, dst_all, acc_sh, *rest) = refs
            rows = rest[:NBUF]
            sems = rest[NBUF:2 * NBUF]
        c = lax.axis_index("c")
        s = lax.axis_index("s")
        base_chunk = jnp.where(c == 0, s * NCH0, 16 * NCH0 + s * NCH1)
        nch = jnp.where(c == 0, NCH0, NCH1)

        row0 = s * STRIPE
        pltpu.sync_copy(zacc.at[pl.ds(row0, STRIPE)],
                        acc_sh.at[pl.ds(row0, STRIPE)])
        if with_deg:
            pltpu.sync_copy(zdeg.at[pl.ds(row0, STRIPE)],
                            deg_sh.at[pl.ds(row0, STRIPE)])
            pltpu.sync_copy(ones, ones_v)
        pltpu.sync_copy(srcp.at[pl.ds(base_chunk, NCH1)],
                        src_all.at[pl.ds(0, NCH1)])
        pltpu.sync_copy(dstp.at[pl.ds(base_chunk, NCH1)],
                        dst_all.at[pl.ds(0, NCH1)])

        @pl.when(c == 0)
        def _stage_rest():
            pltpu.sync_copy(srcp.at[pl.ds(base_chunk + NCH1, NCH0 - NCH1)],
                            src_all.at[pl.ds(NCH1, NCH0 - NCH1)])
            pltpu.sync_copy(dstp.at[pl.ds(base_chunk + NCH1, NCH0 - NCH1)],
                            dst_all.at[pl.ds(NCH1, NCH0 - NCH1)])

        plsc.subcore_barrier()

        for b in range(NBUF):
            pltpu.async_copy(z.at[src_all.at[b]], rows[b], sems[b])

        def group(g, carry2):
            for b in range(NBUF):
                i = g * NBUF + b
                pltpu.make_async_copy(z.at[src_all.at[i]], rows[b],
                                      sems[b]).wait()
                pltpu.sync_copy(rows[b], acc_sh.at[dst_all.at[i]],
                                add=True)
                if with_deg:
                    pltpu.sync_copy(ones_v, deg_sh.at[dst_all.at[i]],
                                    add=True)
                nxt = i + NBUF

                @pl.when(nxt < nch)
                def _prefetch():
                    pltpu.async_copy(z.at[src_all.at[nxt]], rows[b],
                                     sems[b])
            return carry2

        lax.fori_loop(0, nch // NBUF, group, 0)
        plsc.subcore_barrier()

        pltpu.sync_copy(acc_sh.at[pl.ds(row0, STRIPE)],
                        part.at[c, pl.ds(row0, STRIPE)])
        if with_deg:
            pltpu.sync_copy(deg_sh.at[pl.ds(row0, STRIPE)],
                            degpart.at[c, pl.ds(row0, STRIPE)])

    return pl.kernel(body, out_type=tuple(outs), mesh=mesh,
                     scratch_types=tuple(scratch),
                     compiler_params=pltpu.CompilerParams(
                         use_tc_tiling_on_sc=False,
                         needs_layout_passes=False))


_B = 2000


def _tc1(x, wl, wr, b):
    def body(x_ref, wl_ref, wr_ref, b_ref, z_ref, y_ref):
        xb = x_ref[...]
        z_ref[...] = jnp.dot(xb, wl_ref[...], preferred_element_type=jnp.float32)
        y_ref[...] = (jnp.dot(xb, wr_ref[...], preferred_element_type=jnp.float32)
                      + b_ref[0:1, :])
    full = lambda i: (0, 0)
    row = lambda i: (i, 0)
    return pl.pallas_call(
        body,
        grid=(N_NODES // _B,),
        in_specs=[pl.BlockSpec((_B, 128), row), pl.BlockSpec((128, 64), full),
                  pl.BlockSpec((128, 64), full), pl.BlockSpec((8, 64), full)],
        out_specs=[pl.BlockSpec((_B, 64), row), pl.BlockSpec((_B, 64), row)],
        out_shape=[jax.ShapeDtypeStruct((N_NODES, 64), jnp.float32)] * 2,
    )(x, wl, wr, b)


def _tc2(part1, degpart, y1, wl, wr, b):
    def body(p0_ref, p1_ref, d0_ref, d1_ref, y1_ref, wl_ref, wr_ref, b_ref,
             z_ref, y_ref):
        deg = d0_ref[0][:, 0:1] + d1_ref[0][:, 0:1]
        degc = jnp.maximum(deg, 1.0)
        h1 = jnp.maximum((p0_ref[0] + p1_ref[0]) / degc + y1_ref[...], 0.0)
        z_ref[...] = jnp.dot(h1, wl_ref[...], preferred_element_type=jnp.float32)
        y_ref[...] = (jnp.dot(h1, wr_ref[...], preferred_element_type=jnp.float32)
                      + b_ref[0:1, :])
    full = lambda i: (0, 0)
    row = lambda i: (i, 0)
    c0 = lambda i: (0, i, 0)
    c1 = lambda i: (1, i, 0)
    return pl.pallas_call(
        body,
        grid=(N_NODES // _B,),
        in_specs=[pl.BlockSpec((1, _B, 64), c0), pl.BlockSpec((1, _B, 64), c1),
                  pl.BlockSpec((1, _B, 16), c0), pl.BlockSpec((1, _B, 16), c1),
                  pl.BlockSpec((_B, 64), row), pl.BlockSpec((64, 32), full),
                  pl.BlockSpec((64, 32), full), pl.BlockSpec((8, 32), full)],
        out_specs=[pl.BlockSpec((_B, 32), row), pl.BlockSpec((_B, 32), row)],
        out_shape=[jax.ShapeDtypeStruct((N_NODES, 32), jnp.float32),
                   jax.ShapeDtypeStruct((N_NODES, 32), jnp.float32)],
    )(part1, part1, degpart, degpart, y1, wl, wr, b)


def _tc3(part2, degpart, y2, wc1, b1, wc2, b2):
    def body(q0_ref, q1_ref, d0_ref, d1_ref, y2_ref, wc1_ref, b1_ref,
             wc2_ref, b2_ref, out_ref):
        deg = d0_ref[0][:, 0:1] + d1_ref[0][:, 0:1]
        degc = jnp.maximum(deg, 1.0)
        h2 = jnp.maximum((q0_ref[0] + q1_ref[0]) / degc + y2_ref[...], 0.0)
        c1 = jnp.maximum(
            jnp.dot(h2, wc1_ref[...], preferred_element_type=jnp.float32)
            + b1_ref[0:1, :], 0.0)
        out_ref[...] = (jnp.dot(c1, wc2_ref[...], preferred_element_type=jnp.float32)
                        + b2_ref[0:1, :])
    full = lambda i: (0, 0)
    row = lambda i: (i, 0)
    c0 = lambda i: (0, i, 0)
    c1 = lambda i: (1, i, 0)
    return pl.pallas_call(
        body,
        grid=(N_NODES // _B,),
        in_specs=[pl.BlockSpec((1, _B, 32), c0), pl.BlockSpec((1, _B, 32), c1),
                  pl.BlockSpec((1, _B, 16), c0), pl.BlockSpec((1, _B, 16), c1),
                  pl.BlockSpec((_B, 32), row), pl.BlockSpec((32, 16), full),
                  pl.BlockSpec((8, 16), full), pl.BlockSpec((16, 2), full),
                  pl.BlockSpec((8, 2), full)],
        out_specs=pl.BlockSpec((_B, 2), row),
        out_shape=jax.ShapeDtypeStruct((N_NODES, 2), jnp.float32),
    )(part2, part2, degpart, degpart, y2, wc1, b1, wc2, b2)


def kernel(x, edge_index, W1l, W1r, b1, W2l, W2r, b2, Wc1, bc1, Wc2, bc2):
    f32 = jnp.float32
    src = edge_index[0].astype(jnp.int32)
    dst = edge_index[1].astype(jnp.int32)
    pad = NCPAD * C - N_EDGES
    srcp = jnp.concatenate([src, jnp.zeros((pad,), jnp.int32)]
                           ).reshape(NCPAD, C)
    dstp = jnp.concatenate([dst, jnp.full((pad,), N_NODES, jnp.int32)]
                           ).reshape(NCPAD, C)

    zacc64 = jnp.zeros((NP, 64), f32)
    zacc32 = jnp.zeros((NP, 32), f32)
    zdeg = jnp.zeros((NP, 16), f32)
    ones = jnp.ones((C, 16), f32)

    z1, y1 = _tc1(x, W1l.T, W1r.T, jnp.broadcast_to(b1, (8, 64)))

    part1, degpart = _make_sc_seg(64, True)(z1, srcp, dstp, zacc64, zdeg, ones)

    z2, y2 = _tc2(part1, degpart, y1, W2l.T, W2r.T,
                  jnp.broadcast_to(b2, (8, 32)))

    (part2,) = _make_sc_seg(32, False)(z2, srcp, dstp, zacc32)

    out = _tc3(part2, degpart, y2,
               Wc1.T, jnp.broadcast_to(bc1, (8, 16)),
               Wc2.T, jnp.broadcast_to(bc2, (8, 2)))
    return out
